# Initial kernel scaffold; baseline (speedup 1.0000x reference)
import jax
import jax.numpy as jnp
from jax.experimental import pallas as pl


def kernel(x, batch, aggrs, in_proj_weight, in_proj_bias, out_proj_weight, out_proj_bias):
    raise NotImplementedError("write your pallas kernel here")


# speedup vs baseline: 5.5033x; 5.5033x over previous
import functools

import jax
import jax.numpy as jnp
import numpy as np
from jax.experimental import pallas as pl
from jax.experimental.pallas import tpu as pltpu

B = 16
RATIO = 16
HEADS = 8
D = 128
HD = D // HEADS
BLK = 2048
M_INIT = -1e30


def _kernel(off_ref,
            x_ref,
            aggrs_ref,
            inw_ref,
            inb_ref,
            outw_ref,
            outb_ref,
            o_ref,
            mcomb_ref,
            bterm_ref,
            m_ref,
            l_ref,
            acc_ref,
            *, nblocks):
    i = pl.program_id(0)

    @pl.when(i == 0)
    def _init():
        wq = inw_ref[0:D, :]
        q = jax.lax.dot_general(aggrs_ref[...], wq, (((1,), (1,)), ((), ())),
                                preferred_element_type=jnp.float32)
        q = (q + inb_ref[0:1, :]) * (1.0 / np.sqrt(HD))
        qtile = jnp.concatenate([q] * HEADS, axis=0)
        rh = jax.lax.broadcasted_iota(jnp.int32, (D, D), 0) // HD
        ch = jax.lax.broadcasted_iota(jnp.int32, (D, D), 1) // HD
        pmat_t = jnp.where(rh == ch, qtile, 0.0)
        wk = inw_ref[D:2 * D, :]
        mcomb_ref[...] = jax.lax.dot_general(
            pmat_t, wk, (((1,), (0,)), ((), ())),
            preferred_element_type=jnp.float32)
        bk = inb_ref[1:2, :]
        bterm_ref[...] = jax.lax.dot_general(
            pmat_t, bk, (((1,), (1,)), ((), ())),
            preferred_element_type=jnp.float32)
        m_ref[...] = jnp.full((D, B), M_INIT, dtype=jnp.float32)
        l_ref[...] = jnp.zeros((D, B), dtype=jnp.float32)
        acc_ref[...] = jnp.zeros((HEADS, B, RATIO, HD), dtype=jnp.float32)

    x_blk = x_ref[...]
    s = jax.lax.dot_general(mcomb_ref[...], x_blk, (((1,), (1,)), ((), ())),
                            preferred_element_type=jnp.float32)
    s = s + bterm_ref[...]
    wv = inw_ref[2 * D:3 * D, :]
    v = jax.lax.dot_general(x_blk, wv, (((1,), (1,)), ((), ())),
                            preferred_element_type=jnp.float32)
    v = v + inb_ref[2:3, :]

    base = i * BLK
    lane_idx = jax.lax.broadcasted_iota(jnp.int32, (1, BLK), 1) + base

    for b in range(B):
        off_lo = off_ref[b]
        off_hi = off_ref[b + 1]

        @pl.when((off_hi > base) & (off_lo < base + BLK))
        def _seg_update():
            mask = (lane_idx >= off_lo) & (lane_idx < off_hi)
            s_m = jnp.where(mask, s, -jnp.inf)
            m_old = m_ref[:, b:b + 1]
            m_new = jnp.maximum(m_old, jnp.max(s_m, axis=1, keepdims=True))
            corr = jnp.exp(m_old - m_new)
            p = jnp.exp(s_m - m_new)
            l_ref[:, b:b + 1] = l_ref[:, b:b + 1] * corr + jnp.sum(
                p, axis=1, keepdims=True)
            m_ref[:, b:b + 1] = m_new
            for h in range(HEADS):
                p_h = p[h * HD:(h + 1) * HD, :]
                v_h = v[:, h * HD:(h + 1) * HD]
                pv = jax.lax.dot_general(p_h, v_h, (((1,), (0,)), ((), ())),
                                         preferred_element_type=jnp.float32)
                acc_ref[h, b] = (acc_ref[h, b] * corr[h * HD:(h + 1) * HD, :]
                                 + pv)

    @pl.when(i == nblocks - 1)
    def _epilogue():
        rows = []
        for b in range(B):
            l_b = l_ref[:, b:b + 1]
            cols = []
            for h in range(HEADS):
                denom = l_b[h * HD:(h + 1) * HD, :]
                cols.append(acc_ref[h, b] / denom)
            rows.append(jnp.concatenate(cols, axis=1))
        out_pre = jnp.concatenate(rows, axis=0)
        xc = jax.lax.dot_general(out_pre, outw_ref[...],
                                 (((1,), (1,)), ((), ())),
                                 preferred_element_type=jnp.float32)
        xc = xc + outb_ref[...]
        o_ref[...] = xc.reshape(B, RATIO, D)


def kernel(x, batch, aggrs, in_proj_weight, in_proj_bias,
           out_proj_weight, out_proj_bias):
    n = x.shape[0]
    nblocks = (n + BLK - 1) // BLK
    n_pad = nblocks * BLK
    x_pad = jnp.pad(x, ((0, n_pad - n), (0, 0)))
    offs = jnp.searchsorted(batch, jnp.arange(B + 1, dtype=batch.dtype),
                            side="left").astype(jnp.int32)

    grid_spec = pltpu.PrefetchScalarGridSpec(
        num_scalar_prefetch=1,
        grid=(nblocks,),
        in_specs=[
            pl.BlockSpec((BLK, D), lambda i, off: (i, 0)),
            pl.BlockSpec((RATIO, D), lambda i, off: (0, 0)),
            pl.BlockSpec((3 * D, D), lambda i, off: (0, 0)),
            pl.BlockSpec((3, D), lambda i, off: (0, 0)),
            pl.BlockSpec((D, D), lambda i, off: (0, 0)),
            pl.BlockSpec((1, D), lambda i, off: (0, 0)),
        ],
        out_specs=pl.BlockSpec((B, RATIO, D), lambda i, off: (0, 0, 0)),
        scratch_shapes=[
            pltpu.VMEM((D, D), jnp.float32),
            pltpu.VMEM((D, 1), jnp.float32),
            pltpu.VMEM((D, B), jnp.float32),
            pltpu.VMEM((D, B), jnp.float32),
            pltpu.VMEM((HEADS, B, RATIO, HD), jnp.float32),
        ],
    )
    xcent = pl.pallas_call(
        functools.partial(_kernel, nblocks=nblocks),
        grid_spec=grid_spec,
        out_shape=jax.ShapeDtypeStruct((B, RATIO, D), jnp.float32),
    )(offs, x_pad, aggrs,
      in_proj_weight, in_proj_bias.reshape(3, D),
      out_proj_weight, out_proj_bias.reshape(1, D))

    batchcent = jnp.repeat(jnp.arange(B, dtype=jnp.int32), RATIO)
    return (xcent, batchcent)


# speedup vs baseline: 5.6423x; 1.0253x over previous
import functools

import jax
import jax.numpy as jnp
import numpy as np
from jax.experimental import pallas as pl
from jax.experimental.pallas import tpu as pltpu

B = 16
RATIO = 16
HEADS = 8
D = 128
HD = D // HEADS
BLK = 2048
M_INIT = -1e30


def _kernel(off_ref,
            x_ref,
            aggrs_ref,
            inw_ref,
            inb_ref,
            outw_ref,
            outb_ref,
            o_ref,
            mcomb_ref,
            wv_ref,
            bterm_ref,
            m_ref,
            l_ref,
            acc_ref,
            *, nblocks):
    i = pl.program_id(0)

    @pl.when(i == 0)
    def _init():
        wq = inw_ref[0:D, :]
        q = jax.lax.dot_general(aggrs_ref[...], wq, (((1,), (1,)), ((), ())),
                                preferred_element_type=jnp.float32)
        q = (q + inb_ref[0:1, :]) * (1.0 / np.sqrt(HD))
        qtile = jnp.concatenate([q] * HEADS, axis=0)
        rh = jax.lax.broadcasted_iota(jnp.int32, (D, D), 0) // HD
        ch = jax.lax.broadcasted_iota(jnp.int32, (D, D), 1) // HD
        pmat_t = jnp.where(rh == ch, qtile, 0.0)
        wk = inw_ref[D:2 * D, :]
        mcomb_ref[...] = jax.lax.dot_general(
            pmat_t, wk, (((1,), (0,)), ((), ())),
            preferred_element_type=jnp.float32).astype(jnp.bfloat16)
        wv_ref[...] = inw_ref[2 * D:3 * D, :].astype(jnp.bfloat16)
        bk = inb_ref[1:2, :]
        bterm_ref[...] = jax.lax.dot_general(
            pmat_t, bk, (((1,), (1,)), ((), ())),
            preferred_element_type=jnp.float32)
        m_ref[...] = jnp.full((D, B), M_INIT, dtype=jnp.float32)
        l_ref[...] = jnp.zeros((D, B), dtype=jnp.float32)
        acc_ref[...] = jnp.zeros((HEADS, B, RATIO, HD), dtype=jnp.float32)

    x_blk = x_ref[...]
    s = jax.lax.dot_general(mcomb_ref[...], x_blk, (((1,), (1,)), ((), ())),
                            preferred_element_type=jnp.float32)
    s = s + bterm_ref[...]
    v = jax.lax.dot_general(x_blk, wv_ref[...], (((1,), (1,)), ((), ())),
                            preferred_element_type=jnp.float32)
    v = (v + inb_ref[2:3, :]).astype(jnp.bfloat16)

    base = i * BLK
    lane_idx = jax.lax.broadcasted_iota(jnp.int32, (1, BLK), 1) + base

    for b in range(B):
        off_lo = off_ref[b]
        off_hi = off_ref[b + 1]

        @pl.when((off_hi > base) & (off_lo < base + BLK))
        def _seg_update():
            mask = (lane_idx >= off_lo) & (lane_idx < off_hi)
            s_m = jnp.where(mask, s, -jnp.inf)
            m_old = m_ref[:, b:b + 1]
            m_new = jnp.maximum(m_old, jnp.max(s_m, axis=1, keepdims=True))
            corr = jnp.exp(m_old - m_new)
            p = jnp.exp(s_m - m_new)
            l_ref[:, b:b + 1] = l_ref[:, b:b + 1] * corr + jnp.sum(
                p, axis=1, keepdims=True)
            m_ref[:, b:b + 1] = m_new
            p16 = p.astype(jnp.bfloat16)
            for h in range(HEADS):
                p_h = p16[h * HD:(h + 1) * HD, :]
                v_h = v[:, h * HD:(h + 1) * HD]
                pv = jax.lax.dot_general(p_h, v_h, (((1,), (0,)), ((), ())),
                                         preferred_element_type=jnp.float32)
                acc_ref[h, b] = (acc_ref[h, b] * corr[h * HD:(h + 1) * HD, :]
                                 + pv)

    @pl.when(i == nblocks - 1)
    def _epilogue():
        rows = []
        for b in range(B):
            l_b = l_ref[:, b:b + 1]
            cols = []
            for h in range(HEADS):
                denom = l_b[h * HD:(h + 1) * HD, :]
                cols.append(acc_ref[h, b] / denom)
            rows.append(jnp.concatenate(cols, axis=1))
        out_pre = jnp.concatenate(rows, axis=0)
        xc = jax.lax.dot_general(out_pre, outw_ref[...],
                                 (((1,), (1,)), ((), ())),
                                 preferred_element_type=jnp.float32)
        xc = xc + outb_ref[...]
        o_ref[...] = xc.reshape(B, RATIO, D)


def kernel(x, batch, aggrs, in_proj_weight, in_proj_bias,
           out_proj_weight, out_proj_bias):
    n = x.shape[0]
    nblocks = (n + BLK - 1) // BLK
    n_pad = nblocks * BLK
    x_pad = jnp.pad(x, ((0, n_pad - n), (0, 0))).astype(jnp.bfloat16)
    offs = jnp.searchsorted(batch, jnp.arange(B + 1, dtype=batch.dtype),
                            side="left").astype(jnp.int32)

    grid_spec = pltpu.PrefetchScalarGridSpec(
        num_scalar_prefetch=1,
        grid=(nblocks,),
        in_specs=[
            pl.BlockSpec((BLK, D), lambda i, off: (i, 0)),
            pl.BlockSpec((RATIO, D), lambda i, off: (0, 0)),
            pl.BlockSpec((3 * D, D), lambda i, off: (0, 0)),
            pl.BlockSpec((3, D), lambda i, off: (0, 0)),
            pl.BlockSpec((D, D), lambda i, off: (0, 0)),
            pl.BlockSpec((1, D), lambda i, off: (0, 0)),
        ],
        out_specs=pl.BlockSpec((B, RATIO, D), lambda i, off: (0, 0, 0)),
        scratch_shapes=[
            pltpu.VMEM((D, D), jnp.bfloat16),
            pltpu.VMEM((D, D), jnp.bfloat16),
            pltpu.VMEM((D, 1), jnp.float32),
            pltpu.VMEM((D, B), jnp.float32),
            pltpu.VMEM((D, B), jnp.float32),
            pltpu.VMEM((HEADS, B, RATIO, HD), jnp.float32),
        ],
    )
    xcent = pl.pallas_call(
        functools.partial(_kernel, nblocks=nblocks),
        grid_spec=grid_spec,
        out_shape=jax.ShapeDtypeStruct((B, RATIO, D), jnp.float32),
    )(offs, x_pad, aggrs,
      in_proj_weight, in_proj_bias.reshape(3, D),
      out_proj_weight, out_proj_bias.reshape(1, D))

    batchcent = jnp.repeat(jnp.arange(B, dtype=jnp.int32), RATIO)
    return (xcent, batchcent)


# speedup vs baseline: 7.7365x; 1.3712x over previous
import functools

import jax
import jax.numpy as jnp
import numpy as np
from jax.experimental import pallas as pl
from jax.experimental.pallas import tpu as pltpu

B = 16
RATIO = 16
HEADS = 8
D = 128
HD = D // HEADS
BLK = 2048
M_INIT = -1e30


def _kernel(off_ref,
            x_ref,
            aggrs_ref,
            inw_ref,
            inb_ref,
            outw_ref,
            outb_ref,
            o_ref,
            mcomb_ref,
            wv_ref,
            bterm_ref,
            m_ref,
            l_ref,
            acc_ref,
            *, nblocks):
    i = pl.program_id(0)

    @pl.when(i == 0)
    def _init():
        wq = inw_ref[0:D, :]
        q = jax.lax.dot_general(aggrs_ref[...], wq, (((1,), (1,)), ((), ())),
                                preferred_element_type=jnp.float32)
        q = (q + inb_ref[0:1, :]) * (1.0 / np.sqrt(HD))
        qtile = jnp.concatenate([q] * HEADS, axis=0)
        rh = jax.lax.broadcasted_iota(jnp.int32, (D, D), 0) // HD
        ch = jax.lax.broadcasted_iota(jnp.int32, (D, D), 1) // HD
        pmat_t = jnp.where(rh == ch, qtile, 0.0)
        wk = inw_ref[D:2 * D, :]
        mcomb_ref[...] = jax.lax.dot_general(
            pmat_t, wk, (((1,), (0,)), ((), ())),
            preferred_element_type=jnp.float32).astype(jnp.bfloat16)
        wv_ref[...] = inw_ref[2 * D:3 * D, :].T.astype(jnp.bfloat16)
        bk = inb_ref[1:2, :]
        bterm_ref[...] = jax.lax.dot_general(
            pmat_t, bk, (((1,), (1,)), ((), ())),
            preferred_element_type=jnp.float32)
        m_ref[...] = jnp.full((D, B), M_INIT, dtype=jnp.float32)
        l_ref[...] = jnp.zeros((D, B), dtype=jnp.float32)
        acc_ref[...] = jnp.zeros((HEADS, B, RATIO, HD), dtype=jnp.float32)

    x_blk = x_ref[...]
    s = jax.lax.dot_general(mcomb_ref[...], x_blk, (((1,), (1,)), ((), ())),
                            preferred_element_type=jnp.float32)
    s = s + bterm_ref[...]
    v = jax.lax.dot_general(x_blk, wv_ref[...], (((1,), (0,)), ((), ())),
                            preferred_element_type=jnp.float32)
    v = (v + inb_ref[2:3, :]).astype(jnp.bfloat16)
    ones_col = jnp.ones((BLK, 8), dtype=jnp.bfloat16)

    base = i * BLK
    lane_idx = jax.lax.broadcasted_iota(jnp.int32, (1, BLK), 1) + base

    for b in range(B):
        off_lo = off_ref[b]
        off_hi = off_ref[b + 1]

        @pl.when((off_hi > base) & (off_lo < base + BLK))
        def _seg_update():
            mask = (lane_idx >= off_lo) & (lane_idx < off_hi)
            s_m = jnp.where(mask, s, -jnp.inf)
            m_old = m_ref[:, b:b + 1]
            m_new = jnp.maximum(m_old, jnp.max(s_m, axis=1, keepdims=True))
            corr = jnp.exp(m_old - m_new)
            p16 = jnp.exp(s_m - m_new).astype(jnp.bfloat16)
            l_add = jax.lax.dot_general(p16, ones_col,
                                        (((1,), (0,)), ((), ())),
                                        preferred_element_type=jnp.float32)
            l_ref[:, b:b + 1] = l_ref[:, b:b + 1] * corr + l_add[:, 0:1]
            m_ref[:, b:b + 1] = m_new
            pv_full = jax.lax.dot_general(p16, v, (((1,), (0,)), ((), ())),
                                          preferred_element_type=jnp.float32)
            for h in range(HEADS):
                acc_ref[h, b] = (acc_ref[h, b] * corr[h * HD:(h + 1) * HD, :]
                                 + pv_full[h * HD:(h + 1) * HD,
                                           h * HD:(h + 1) * HD])

    @pl.when(i == nblocks - 1)
    def _epilogue():
        rows = []
        for b in range(B):
            l_b = l_ref[:, b:b + 1]
            cols = []
            for h in range(HEADS):
                denom = l_b[h * HD:(h + 1) * HD, :]
                cols.append(acc_ref[h, b] / denom)
            rows.append(jnp.concatenate(cols, axis=1))
        out_pre = jnp.concatenate(rows, axis=0)
        xc = jax.lax.dot_general(out_pre, outw_ref[...],
                                 (((1,), (1,)), ((), ())),
                                 preferred_element_type=jnp.float32)
        xc = xc + outb_ref[...]
        o_ref[...] = xc.reshape(B, RATIO, D)


def kernel(x, batch, aggrs, in_proj_weight, in_proj_bias,
           out_proj_weight, out_proj_bias):
    n = x.shape[0]
    nblocks = (n + BLK - 1) // BLK
    n_pad = nblocks * BLK
    x_pad = jnp.pad(x, ((0, n_pad - n), (0, 0))).astype(jnp.bfloat16)
    offs = jnp.searchsorted(batch, jnp.arange(B + 1, dtype=batch.dtype),
                            side="left").astype(jnp.int32)

    grid_spec = pltpu.PrefetchScalarGridSpec(
        num_scalar_prefetch=1,
        grid=(nblocks,),
        in_specs=[
            pl.BlockSpec((BLK, D), lambda i, off: (i, 0)),
            pl.BlockSpec((RATIO, D), lambda i, off: (0, 0)),
            pl.BlockSpec((3 * D, D), lambda i, off: (0, 0)),
            pl.BlockSpec((3, D), lambda i, off: (0, 0)),
            pl.BlockSpec((D, D), lambda i, off: (0, 0)),
            pl.BlockSpec((1, D), lambda i, off: (0, 0)),
        ],
        out_specs=pl.BlockSpec((B, RATIO, D), lambda i, off: (0, 0, 0)),
        scratch_shapes=[
            pltpu.VMEM((D, D), jnp.bfloat16),
            pltpu.VMEM((D, D), jnp.bfloat16),
            pltpu.VMEM((D, 1), jnp.float32),
            pltpu.VMEM((D, B), jnp.float32),
            pltpu.VMEM((D, B), jnp.float32),
            pltpu.VMEM((HEADS, B, RATIO, HD), jnp.float32),
        ],
    )
    xcent = pl.pallas_call(
        functools.partial(_kernel, nblocks=nblocks),
        grid_spec=grid_spec,
        out_shape=jax.ShapeDtypeStruct((B, RATIO, D), jnp.float32),
    )(offs, x_pad, aggrs,
      in_proj_weight, in_proj_bias.reshape(3, D),
      out_proj_weight, out_proj_bias.reshape(1, D))

    batchcent = jnp.repeat(jnp.arange(B, dtype=jnp.int32), RATIO)
    return (xcent, batchcent)
